# baseline (speedup 1.0000x reference)
import functools

import jax
import jax.numpy as jnp
from jax import lax
from jax.experimental import pallas as pl
from jax.experimental.pallas import tpu as pltpu
from jax.experimental.pallas import tpu_sc as plsc

VOCAB = 100000
EMBED_DIM = 50
PAD_DIM = 64
BATCH = 4096
SEQ = 50

_B = BATCH * SEQ
_NC, _NS = 2, 16
_NW = _NC * _NS
_BPW = BATCH // _NW
_PER_W = _B // _NW
_CHUNK = 100
_SUPER = 4
_NSUPER = _BPW // _SUPER


def _repack_half(raw, pk, half):
    def body(s, _):
        for b, roff in ((2 * half, 0), (2 * half + 1, EMBED_DIM)):
            r = s + roff
            pbase = b * (SEQ * EMBED_DIM) + s * EMBED_DIM
            for c in (0, 16, 32):
                pk[pl.ds(pbase + c, 16)] = raw[r, pl.ds(c, 16)]
            pk[pl.ds(pbase + EMBED_DIM - 16, 16)] = raw[r, pl.ds(EMBED_DIM - 16, 16)]
        return _

    lax.fori_loop(0, SEQ, body, 0)


_DEPTH = 3
_NCHUNK = _PER_W // _CHUNK


def _gather_body(
    table_hbm, idx_hbm, out_hbm,
    idx_v, stage_a, stage_b, stage_c, raw_a, raw_b, raw_c, pk_a, pk_b,
    gsem_a, gsem_b, gsem_c, osem_a, osem_b,
):
    wid = lax.axis_index("c") * _NS + lax.axis_index("s")
    base = wid * _PER_W
    pltpu.sync_copy(idx_hbm.at[pl.ds(base, _PER_W)], idx_v.at[pl.ds(0, _PER_W)])

    ivec = lax.iota(jnp.int32, 16)
    stages = (stage_a, stage_b, stage_c)
    raws = (raw_a, raw_b, raw_c)
    pks = (pk_a, pk_b)
    gsems = (gsem_a, gsem_b, gsem_c)
    osems = (osem_a, osem_b)

    def fire(ch):
        if ch >= _NCHUNK:
            return
        st = stages[ch % _DEPTH]
        off = ch * _CHUNK
        for t in range(7):
            st[pl.ds(t * 16, 16)] = plsc.load_gather(idx_v, [ivec + (off + t * 16)])
        pltpu.async_copy(
            table_hbm.at[st.at[pl.ds(0, _CHUNK)]],
            raws[ch % _DEPTH],
            gsems[ch % _DEPTH],
        )

    def wait_gather(ch):
        pltpu.make_async_copy(
            table_hbm.at[stages[ch % _DEPTH].at[pl.ds(0, _CHUNK)]],
            raws[ch % _DEPTH],
            gsems[ch % _DEPTH],
        ).wait()

    _BLK = _SUPER * SEQ * EMBED_DIM

    def out_copy(k):
        return (
            pks[k % 2],
            out_hbm.at[pl.ds((wid * _BPW + k * _SUPER) * SEQ * EMBED_DIM, _BLK)],
            osems[k % 2],
        )

    fire(0)
    fire(1)
    fire(2)
    for k in range(_NSUPER):
        wait_gather(2 * k)
        if k >= 2:
            src, dst, sem = out_copy(k - 2)
            pltpu.make_async_copy(src, dst, sem).wait()
        _repack_half(raws[(2 * k) % _DEPTH], pks[k % 2], 0)
        fire(2 * k + 3)
        wait_gather(2 * k + 1)
        _repack_half(raws[(2 * k + 1) % _DEPTH], pks[k % 2], 1)
        fire(2 * k + 4)
        src, dst, sem = out_copy(k)
        pltpu.async_copy(src, dst, sem)
    for k in (_NSUPER - 2, _NSUPER - 1):
        src, dst, sem = out_copy(k)
        pltpu.make_async_copy(src, dst, sem).wait()


@jax.jit
def _embed_gather(table_padded, idx_flat):
    k = functools.partial(
        pl.kernel,
        out_type=jax.ShapeDtypeStruct((_B * EMBED_DIM,), jnp.float32),
        mesh=plsc.VectorSubcoreMesh(core_axis_name="c", subcore_axis_name="s"),
        scratch_types=[
            pltpu.VMEM((_PER_W + 16,), jnp.int32),
            pltpu.VMEM((112,), jnp.int32),
            pltpu.VMEM((112,), jnp.int32),
            pltpu.VMEM((112,), jnp.int32),
            pltpu.VMEM((_CHUNK, PAD_DIM), jnp.float32),
            pltpu.VMEM((_CHUNK, PAD_DIM), jnp.float32),
            pltpu.VMEM((_CHUNK, PAD_DIM), jnp.float32),
            pltpu.VMEM((_SUPER * SEQ * EMBED_DIM,), jnp.float32),
            pltpu.VMEM((_SUPER * SEQ * EMBED_DIM,), jnp.float32),
            pltpu.SemaphoreType.DMA,
            pltpu.SemaphoreType.DMA,
            pltpu.SemaphoreType.DMA,
            pltpu.SemaphoreType.DMA,
            pltpu.SemaphoreType.DMA,
        ],
        compiler_params=pltpu.CompilerParams(
            use_tc_tiling_on_sc=False, needs_layout_passes=False
        ),
    )(_gather_body)
    return k(table_padded, idx_flat)


def kernel(indices, table):
    table_padded = jnp.pad(table, ((0, 0), (0, PAD_DIM - EMBED_DIM)))
    idx_flat = indices.reshape(_B)
    out = _embed_gather(table_padded, idx_flat)
    return out.reshape(BATCH, SEQ, EMBED_DIM)


# speedup vs baseline: 1.1448x; 1.1448x over previous
import functools

import jax
import jax.numpy as jnp
from jax import lax
from jax.experimental import pallas as pl
from jax.experimental.pallas import tpu as pltpu
from jax.experimental.pallas import tpu_sc as plsc

VOCAB = 100000
EMBED_DIM = 50
PAD_DIM = 64
BATCH = 4096
SEQ = 50

_B = BATCH * SEQ
_NC, _NS = 2, 16
_NW = _NC * _NS
_BPW = BATCH // _NW
_PER_W = _B // _NW
_CHUNK = 100
_SUPER = 4
_NSUPER = _BPW // _SUPER


def _repack_half(raw, pk, half):
    def body(s, _):
        for b, roff in ((2 * half, 0), (2 * half + 1, EMBED_DIM)):
            r = s + roff
            for c in (0, 16, 32):
                pk[b, s, pl.ds(c, 16)] = raw[r, pl.ds(c, 16)]
            pk[b, s, pl.ds(EMBED_DIM - 16, 16)] = raw[r, pl.ds(EMBED_DIM - 16, 16)]
        return _

    lax.fori_loop(0, SEQ, body, 0)


_DEPTH = 3
_NCHUNK = _PER_W // _CHUNK


def _gather_body(
    table_hbm, idx_hbm, out_hbm,
    idx_v, stage_a, stage_b, stage_c, raw_a, raw_b, raw_c, pk_a, pk_b,
    gsem_a, gsem_b, gsem_c, osem_a, osem_b,
):
    wid = lax.axis_index("c") * _NS + lax.axis_index("s")
    base = wid * _PER_W
    pltpu.sync_copy(idx_hbm.at[pl.ds(base, _PER_W)], idx_v.at[pl.ds(0, _PER_W)])

    ivec = lax.iota(jnp.int32, 16)
    stages = (stage_a, stage_b, stage_c)
    raws = (raw_a, raw_b, raw_c)
    pks = (pk_a, pk_b)
    gsems = (gsem_a, gsem_b, gsem_c)
    osems = (osem_a, osem_b)

    def fire(ch):
        if ch >= _NCHUNK:
            return
        st = stages[ch % _DEPTH]
        off = ch * _CHUNK
        for t in range(7):
            st[pl.ds(t * 16, 16)] = plsc.load_gather(idx_v, [ivec + (off + t * 16)])
        pltpu.async_copy(
            table_hbm.at[st.at[pl.ds(0, _CHUNK)]],
            raws[ch % _DEPTH],
            gsems[ch % _DEPTH],
        )

    def wait_gather(ch):
        pltpu.make_async_copy(
            table_hbm.at[stages[ch % _DEPTH].at[pl.ds(0, _CHUNK)]],
            raws[ch % _DEPTH],
            gsems[ch % _DEPTH],
        ).wait()

    def out_copy(k):
        return (
            pks[k % 2],
            out_hbm.at[pl.ds(wid * _BPW + k * _SUPER, _SUPER)],
            osems[k % 2],
        )

    fire(0)
    fire(1)
    fire(2)
    for k in range(_NSUPER):
        wait_gather(2 * k)
        if k >= 2:
            src, dst, sem = out_copy(k - 2)
            pltpu.make_async_copy(src, dst, sem).wait()
        _repack_half(raws[(2 * k) % _DEPTH], pks[k % 2], 0)
        fire(2 * k + 3)
        wait_gather(2 * k + 1)
        _repack_half(raws[(2 * k + 1) % _DEPTH], pks[k % 2], 1)
        fire(2 * k + 4)
        src, dst, sem = out_copy(k)
        pltpu.async_copy(src, dst, sem)
    for k in (_NSUPER - 2, _NSUPER - 1):
        src, dst, sem = out_copy(k)
        pltpu.make_async_copy(src, dst, sem).wait()


@jax.jit
def _embed_gather(table_padded, idx_flat):
    k = functools.partial(
        pl.kernel,
        out_type=jax.ShapeDtypeStruct((BATCH, SEQ, EMBED_DIM), jnp.float32),
        mesh=plsc.VectorSubcoreMesh(core_axis_name="c", subcore_axis_name="s"),
        scratch_types=[
            pltpu.VMEM((_PER_W + 16,), jnp.int32),
            pltpu.VMEM((112,), jnp.int32),
            pltpu.VMEM((112,), jnp.int32),
            pltpu.VMEM((112,), jnp.int32),
            pltpu.VMEM((_CHUNK, PAD_DIM), jnp.float32),
            pltpu.VMEM((_CHUNK, PAD_DIM), jnp.float32),
            pltpu.VMEM((_CHUNK, PAD_DIM), jnp.float32),
            pltpu.VMEM((_SUPER, SEQ, EMBED_DIM), jnp.float32),
            pltpu.VMEM((_SUPER, SEQ, EMBED_DIM), jnp.float32),
            pltpu.SemaphoreType.DMA,
            pltpu.SemaphoreType.DMA,
            pltpu.SemaphoreType.DMA,
            pltpu.SemaphoreType.DMA,
            pltpu.SemaphoreType.DMA,
        ],
        compiler_params=pltpu.CompilerParams(
            use_tc_tiling_on_sc=False, needs_layout_passes=False
        ),
    )(_gather_body)
    return k(table_padded, idx_flat)


def kernel(indices, table):
    table_padded = jnp.pad(table, ((0, 0), (0, PAD_DIM - EMBED_DIM)))
    idx_flat = indices.reshape(_B)
    return _embed_gather(table_padded, idx_flat)
